# baseline (speedup 1.0000x reference)
import jax
import jax.numpy as jnp
import numpy as np
from jax import lax
from jax.experimental import pallas as pl
from jax.experimental.pallas import tpu as pltpu
from jax.experimental.pallas import tpu_sc as plsc

NUM_ATTR = 8
NUM_CLASSES = 1000
TOTAL_DIM = 3072
DIMS_PER_ATTR = TOTAL_DIM // NUM_ATTR
BATCH = 4096

NB_SC = 2048
NB_TC = BATCH - NB_SC
BB = 512

NC, NS, L = 2, 16, 16
NW = NC * NS
RPW = NB_SC // NW
CH = 8
NCHUNK = RPW // CH
GROWS = CH * NUM_ATTR


def _sc_body(z_hbm, lab_hbm, means_hbm, out_hbm,
             lab_v, idx_v, z0, z1, g0, g1, acc_v,
             sz0, sz1, sg0, sg1):
    wid = lax.axis_index("s") * NC + lax.axis_index("c")
    row0 = wid * RPW
    bufs = ((z0, g0, sz0, sg0), (z1, g1, sz1, sg1))

    pltpu.sync_copy(lab_hbm.at[pl.ds(row0 * NUM_ATTR, RPW * NUM_ATTR)], lab_v)
    offs = (lax.iota(jnp.int32, L) % NUM_ATTR) * NUM_CLASSES

    def mkidx(j, carry):
        idx_v[pl.ds(j * L, L)] = lab_v[pl.ds(j * L, L)] + offs
        return carry

    lax.fori_loop(0, RPW * NUM_ATTR // L, mkidx, 0)

    def start(c, b):
        zb, gb, zsem, gsem = bufs[b]
        dz = pltpu.async_copy(
            z_hbm.at[pl.ds(row0 + c * CH, CH), :], zb, zsem)
        dg = pltpu.async_copy(
            means_hbm.at[idx_v.at[pl.ds(c * GROWS, GROWS)]], gb, gsem)
        return dz, dg

    pend = [start(0, 0), None]

    NACC = 8
    accs = tuple(jnp.zeros((L,), jnp.float32) for _ in range(NACC))
    for c in range(NCHUNK):
        b = c & 1
        if c + 1 < NCHUNK:
            pend[b ^ 1] = start(c + 1, b ^ 1)
        dz, dg = pend[b]
        dz.wait()
        dg.wait()
        zb, gb = bufs[b][0], bufs[b][1]

        def row(g, a, zb=zb, gb=gb):
            a = list(a)
            r = g // NUM_ATTR
            col0 = (g % NUM_ATTR) * DIMS_PER_ATTR
            for s in range(DIMS_PER_ATTR // (2 * L)):
                mi = gb[g, pl.ds(s * L, L)]
                m0 = lax.bitcast_convert_type(mi << 16, jnp.float32)
                m1 = lax.bitcast_convert_type(mi & jnp.int32(-65536),
                                              jnp.float32)
                d0 = zb[r, pl.ds(col0 + 2 * s * L, L)] - m0
                d1 = zb[r, pl.ds(col0 + (2 * s + 1) * L, L)] - m1
                a[(2 * s) % NACC] = a[(2 * s) % NACC] + d0 * d0
                a[(2 * s + 1) % NACC] = a[(2 * s + 1) % NACC] + d1 * d1
            return tuple(a)

        accs = lax.fori_loop(0, GROWS, row, accs)

    acc = ((accs[0] + accs[1]) + (accs[2] + accs[3])) + \
          ((accs[4] + accs[5]) + (accs[6] + accs[7]))
    acc_v[...] = 0.5 * acc
    pltpu.sync_copy(acc_v, out_hbm.at[wid])


def _tc_body(z_ref, sldj_ref, lab_ref, means_ref, out_ref):
    pid = pl.program_id(0)

    @pl.when(pid == 0)
    def _init():
        out_ref[...] = jnp.reshape(-jnp.sum(sldj_ref[...]), (1, 1))

    z = z_ref[...]
    acc = 0.5 * jnp.sum(z * z)
    labs = lab_ref[...]
    class_iota = lax.broadcasted_iota(jnp.int32, (BB, NUM_CLASSES), 1)
    for i in range(NUM_ATTR):
        onehot = (labs[:, i][:, None] == class_iota).astype(jnp.bfloat16)
        sel = jnp.dot(onehot, means_ref[i],
                      preferred_element_type=jnp.float32)
        zseg = z[:, i * DIMS_PER_ATTR:(i + 1) * DIMS_PER_ATTR]
        acc += 0.5 * jnp.sum(sel * sel) - jnp.sum(zseg * sel)
    out_ref[...] += jnp.reshape(acc, (1, 1))


@jax.jit
def kernel(z, sldj, labels, means):
    labels = labels.astype(jnp.int32)
    lab_flat = labels.reshape(BATCH * NUM_ATTR)
    R = NUM_ATTR * NUM_CLASSES
    mperm = (means.reshape(R, DIMS_PER_ATTR // 32, 2, 16)
             .transpose(0, 1, 3, 2)
             .astype(jnp.bfloat16))
    mpacked = lax.bitcast_convert_type(mperm, jnp.int32).reshape(
        R, DIMS_PER_ATTR // 2)
    mpacked = jnp.concatenate(
        [mpacked, jnp.zeros((R, 256 - DIMS_PER_ATTR // 2), jnp.int32)], axis=1)

    mesh = plsc.VectorSubcoreMesh(core_axis_name="c", subcore_axis_name="s")
    sc_out = pl.kernel(
        _sc_body,
        out_type=jax.ShapeDtypeStruct((NW, L), jnp.float32),
        mesh=mesh,
        scratch_types=[
            pltpu.VMEM((RPW * NUM_ATTR,), jnp.int32),
            pltpu.VMEM((RPW * NUM_ATTR,), jnp.int32),
            pltpu.VMEM((CH, TOTAL_DIM), jnp.float32),
            pltpu.VMEM((CH, TOTAL_DIM), jnp.float32),
            pltpu.VMEM((GROWS, 256), jnp.int32),
            pltpu.VMEM((GROWS, 256), jnp.int32),
            pltpu.VMEM((L,), jnp.float32),
            pltpu.SemaphoreType.DMA,
            pltpu.SemaphoreType.DMA,
            pltpu.SemaphoreType.DMA,
            pltpu.SemaphoreType.DMA,
        ],
    )(z, lab_flat, mpacked)

    means_bf = means.astype(jnp.bfloat16)
    sldj2d = sldj.reshape(32, BATCH // 32)
    grid = NB_TC // BB
    off = NB_SC // BB
    tc_out = pl.pallas_call(
        _tc_body,
        grid=(grid,),
        in_specs=[
            pl.BlockSpec((BB, TOTAL_DIM), lambda b: (b + off, 0)),
            pl.BlockSpec((32, BATCH // 32), lambda b: (0, 0)),
            pl.BlockSpec((BB, NUM_ATTR), lambda b: (b + off, 0)),
            pl.BlockSpec((NUM_ATTR, NUM_CLASSES, DIMS_PER_ATTR),
                         lambda b: (0, 0, 0)),
        ],
        out_specs=pl.BlockSpec((1, 1), lambda b: (0, 0)),
        out_shape=jax.ShapeDtypeStruct((1, 1), jnp.float32),
    )(z, sldj2d, labels, means_bf)

    total = jnp.sum(sc_out) + tc_out[0, 0]
    const = 0.5 * TOTAL_DIM * np.log(2 * np.pi)
    return total / BATCH + jnp.float32(const)


# speedup vs baseline: 1.6641x; 1.6641x over previous
import jax
import jax.numpy as jnp
import numpy as np
from jax import lax
from jax.experimental import pallas as pl
from jax.experimental.pallas import tpu as pltpu
from jax.experimental.pallas import tpu_sc as plsc

NUM_ATTR = 8
NUM_CLASSES = 1000
TOTAL_DIM = 3072
DIMS_PER_ATTR = TOTAL_DIM // NUM_ATTR
BATCH = 4096

NB_SC = 2048
NB_TC = BATCH - NB_SC
BB = 512

NC, NS, L = 2, 16, 16
NW = NC * NS
RPW = NB_SC // NW
CH = 8
NCHUNK = RPW // CH
GROWS = CH * NUM_ATTR


def _sc_body(z_hbm, lab_hbm, means_hbm, out_hbm,
             lab_v, idx_v, z0, z1, g0, g1, acc_v,
             sz0, sz1, sg0, sg1):
    wid = lax.axis_index("s") * NC + lax.axis_index("c")
    row0 = wid * RPW
    bufs = ((z0, g0, sz0, sg0), (z1, g1, sz1, sg1))

    pltpu.sync_copy(lab_hbm.at[pl.ds(row0 * NUM_ATTR, RPW * NUM_ATTR)], lab_v)
    offs = (lax.iota(jnp.int32, L) % NUM_ATTR) * NUM_CLASSES

    def mkidx(j, carry):
        idx_v[pl.ds(j * L, L)] = lab_v[pl.ds(j * L, L)] + offs
        return carry

    lax.fori_loop(0, RPW * NUM_ATTR // L, mkidx, 0)

    def start(c, b):
        zb, gb, zsem, gsem = bufs[b]
        dz = pltpu.async_copy(
            z_hbm.at[pl.ds(row0 + c * CH, CH), :], zb, zsem)
        dg = pltpu.async_copy(
            means_hbm.at[idx_v.at[pl.ds(c * GROWS, GROWS)]], gb, gsem)
        return dz, dg

    pend = [start(0, 0), None]

    NACC = 8
    accs = tuple(jnp.zeros((L,), jnp.float32) for _ in range(NACC))
    for c in range(NCHUNK):
        b = c & 1
        if c + 1 < NCHUNK:
            pend[b ^ 1] = start(c + 1, b ^ 1)
        dz, dg = pend[b]
        dz.wait()
        dg.wait()
        zb, gb = bufs[b][0], bufs[b][1]

        def row(g, a, zb=zb, gb=gb):
            a = list(a)
            r = g // NUM_ATTR
            col0 = (g % NUM_ATTR) * DIMS_PER_ATTR
            for s in range(DIMS_PER_ATTR // (2 * L)):
                mi = gb[g, pl.ds(s * L, L)]
                m0 = lax.bitcast_convert_type(mi << 16, jnp.float32)
                m1 = lax.bitcast_convert_type(mi & jnp.int32(-65536),
                                              jnp.float32)
                d0 = zb[r, pl.ds(col0 + s * L, L)] - m0
                d1 = zb[r, pl.ds(col0 + DIMS_PER_ATTR // 2 + s * L, L)] - m1
                a[(2 * s) % NACC] = a[(2 * s) % NACC] + d0 * d0
                a[(2 * s + 1) % NACC] = a[(2 * s + 1) % NACC] + d1 * d1
            return tuple(a)

        accs = lax.fori_loop(0, GROWS, row, accs)

    acc = ((accs[0] + accs[1]) + (accs[2] + accs[3])) + \
          ((accs[4] + accs[5]) + (accs[6] + accs[7]))
    acc_v[...] = 0.5 * acc
    pltpu.sync_copy(acc_v, out_hbm.at[wid])


def _tc_body(z_ref, sldj_ref, lab_ref, means_ref, out_ref):
    pid = pl.program_id(0)

    @pl.when(pid == 0)
    def _init():
        out_ref[...] = jnp.reshape(-jnp.sum(sldj_ref[...]), (1, 1))

    z = z_ref[...]
    acc = 0.5 * jnp.sum(z * z)
    labs = lab_ref[...]
    class_iota = lax.broadcasted_iota(jnp.int32, (BB, NUM_CLASSES), 1)
    for i in range(NUM_ATTR):
        onehot = (labs[:, i][:, None] == class_iota).astype(jnp.bfloat16)
        sel = jnp.dot(onehot, means_ref[i],
                      preferred_element_type=jnp.float32)
        zseg = z[:, i * DIMS_PER_ATTR:(i + 1) * DIMS_PER_ATTR]
        acc += 0.5 * jnp.sum(sel * sel) - jnp.sum(zseg * sel)
    out_ref[...] += jnp.reshape(acc, (1, 1))


@jax.jit
def kernel(z, sldj, labels, means):
    labels = labels.astype(jnp.int32)
    lab_flat = labels.reshape(BATCH * NUM_ATTR)
    R = NUM_ATTR * NUM_CLASSES
    HALF = DIMS_PER_ATTR // 2
    bits = lax.bitcast_convert_type(
        means.reshape(R, DIMS_PER_ATTR), jnp.int32)
    rnd = jnp.int32(0x8000)
    lo = ((bits[:, :HALF] + rnd) >> 16) & jnp.int32(0xFFFF)
    hi = (bits[:, HALF:] + rnd) & jnp.int32(-65536)
    mpacked = hi | lo
    mpacked = jnp.concatenate(
        [mpacked, jnp.zeros((R, 256 - HALF), jnp.int32)], axis=1)

    mesh = plsc.VectorSubcoreMesh(core_axis_name="c", subcore_axis_name="s")
    sc_out = pl.kernel(
        _sc_body,
        out_type=jax.ShapeDtypeStruct((NW, L), jnp.float32),
        mesh=mesh,
        scratch_types=[
            pltpu.VMEM((RPW * NUM_ATTR,), jnp.int32),
            pltpu.VMEM((RPW * NUM_ATTR,), jnp.int32),
            pltpu.VMEM((CH, TOTAL_DIM), jnp.float32),
            pltpu.VMEM((CH, TOTAL_DIM), jnp.float32),
            pltpu.VMEM((GROWS, 256), jnp.int32),
            pltpu.VMEM((GROWS, 256), jnp.int32),
            pltpu.VMEM((L,), jnp.float32),
            pltpu.SemaphoreType.DMA,
            pltpu.SemaphoreType.DMA,
            pltpu.SemaphoreType.DMA,
            pltpu.SemaphoreType.DMA,
        ],
    )(z, lab_flat, mpacked)

    means_bf = means.astype(jnp.bfloat16)
    sldj2d = sldj.reshape(32, BATCH // 32)
    grid = NB_TC // BB
    off = NB_SC // BB
    tc_out = pl.pallas_call(
        _tc_body,
        grid=(grid,),
        in_specs=[
            pl.BlockSpec((BB, TOTAL_DIM), lambda b: (b + off, 0)),
            pl.BlockSpec((32, BATCH // 32), lambda b: (0, 0)),
            pl.BlockSpec((BB, NUM_ATTR), lambda b: (b + off, 0)),
            pl.BlockSpec((NUM_ATTR, NUM_CLASSES, DIMS_PER_ATTR),
                         lambda b: (0, 0, 0)),
        ],
        out_specs=pl.BlockSpec((1, 1), lambda b: (0, 0)),
        out_shape=jax.ShapeDtypeStruct((1, 1), jnp.float32),
    )(z, sldj2d, labels, means_bf)

    total = jnp.sum(sc_out) + tc_out[0, 0]
    const = 0.5 * TOTAL_DIM * np.log(2 * np.pi)
    return total / BATCH + jnp.float32(const)


# speedup vs baseline: 2.2350x; 1.3430x over previous
import jax
import jax.numpy as jnp
import numpy as np
from jax import lax
from jax.experimental import pallas as pl
from jax.experimental.pallas import tpu as pltpu
from jax.experimental.pallas import tpu_sc as plsc

NUM_ATTR = 8
NUM_CLASSES = 1000
TOTAL_DIM = 3072
DIMS_PER_ATTR = TOTAL_DIM // NUM_ATTR
BATCH = 4096

NB_SC = 2048
NB_TC = BATCH - NB_SC
BB = 512

NC, NS, L = 2, 16, 16
NW = NC * NS
RPW = NB_SC // NW
CH = 8
NCHUNK = RPW // CH
GROWS = CH * NUM_ATTR


def _sc_body(z_hbm, lab_hbm, means_hbm, out_hbm,
             lab_v, idx_v, z0, z1, g0, g1, acc_v,
             sz0, sz1, sg0, sg1):
    wid = lax.axis_index("s") * NC + lax.axis_index("c")
    row0 = wid * RPW
    bufs = ((z0, g0, sz0, sg0), (z1, g1, sz1, sg1))

    pltpu.sync_copy(lab_hbm.at[pl.ds(row0 * NUM_ATTR, RPW * NUM_ATTR)], lab_v)
    offs = (lax.iota(jnp.int32, L) % NUM_ATTR) * NUM_CLASSES

    def mkidx(j, carry):
        idx_v[pl.ds(j * L, L)] = lab_v[pl.ds(j * L, L)] + offs
        return carry

    lax.fori_loop(0, RPW * NUM_ATTR // L, mkidx, 0)

    def start(c, b):
        zb, gb, zsem, gsem = bufs[b]
        dz = pltpu.async_copy(
            z_hbm.at[pl.ds(row0 + c * CH, CH), :], zb, zsem)
        dg = pltpu.async_copy(
            means_hbm.at[idx_v.at[pl.ds(c * GROWS, GROWS)]], gb, gsem)
        return dz, dg

    pend = [start(0, 0), None]

    NACC = 8
    accs = tuple(jnp.zeros((L,), jnp.float32) for _ in range(NACC))
    for c in range(NCHUNK):
        b = c & 1
        if c + 1 < NCHUNK:
            pend[b ^ 1] = start(c + 1, b ^ 1)
        dz, dg = pend[b]
        dz.wait()
        dg.wait()
        zb, gb = bufs[b][0], bufs[b][1]

        def row(g, a, zb=zb, gb=gb):
            a = list(a)
            r = g // NUM_ATTR
            col0 = (g % NUM_ATTR) * DIMS_PER_ATTR
            for s in range(8):
                mi = gb[g, pl.ds(s * L, L)]
                m0 = lax.bitcast_convert_type(mi << 16, jnp.float32)
                m1 = lax.bitcast_convert_type(mi & jnp.int32(-65536),
                                              jnp.float32)
                d0 = zb[r, pl.ds(col0 + s * L, L)] - m0
                d1 = zb[r, pl.ds(col0 + 128 + s * L, L)] - m1
                a[(2 * s) % NACC] = a[(2 * s) % NACC] + d0 * d0
                a[(2 * s + 1) % NACC] = a[(2 * s + 1) % NACC] + d1 * d1
            for s in range(8):
                mi = gb[g, pl.ds(128 + s * L, L)]
                m2 = lax.bitcast_convert_type(mi << 16, jnp.float32)
                d2 = zb[r, pl.ds(col0 + 256 + s * L, L)] - m2
                a[s % NACC] = a[s % NACC] + d2 * d2
            return tuple(a)

        accs = lax.fori_loop(0, GROWS, row, accs)

    acc = ((accs[0] + accs[1]) + (accs[2] + accs[3])) + \
          ((accs[4] + accs[5]) + (accs[6] + accs[7]))
    acc_v[...] = 0.5 * acc
    pltpu.sync_copy(acc_v, out_hbm.at[wid])


def _prep_body(m_ref, mp_ref, mb_ref):
    m = m_ref[0]
    bits = lax.bitcast_convert_type(m, jnp.int32)
    r = jnp.int32(0x8000)
    b0 = bits[:, 0:128] + r
    b1 = bits[:, 128:256] + r
    b2 = bits[:, 256:384] + r
    w01 = ((b0 >> 16) & jnp.int32(0xFFFF)) | (b1 & jnp.int32(-65536))
    w2 = (b2 >> 16) & jnp.int32(0xFFFF)
    mp_ref[0] = jnp.concatenate([w01, w2], axis=1)
    mb_ref[0] = m.astype(jnp.bfloat16)


def _tc_body(z_ref, sldj_ref, lab_ref, means_ref, out_ref):
    pid = pl.program_id(0)

    @pl.when(pid == 0)
    def _init():
        out_ref[...] = jnp.reshape(-jnp.sum(sldj_ref[...]), (1, 1))

    z = z_ref[...]
    acc = 0.5 * jnp.sum(z * z)
    labs = lab_ref[...]
    class_iota = lax.broadcasted_iota(jnp.int32, (BB, NUM_CLASSES), 1)
    for i in range(NUM_ATTR):
        onehot = (labs[:, i][:, None] == class_iota).astype(jnp.bfloat16)
        sel = jnp.dot(onehot, means_ref[i],
                      preferred_element_type=jnp.float32)
        zseg = z[:, i * DIMS_PER_ATTR:(i + 1) * DIMS_PER_ATTR]
        acc += 0.5 * jnp.sum(sel * sel) - jnp.sum(zseg * sel)
    out_ref[...] += jnp.reshape(acc, (1, 1))


@jax.jit
def kernel(z, sldj, labels, means):
    labels = labels.astype(jnp.int32)
    lab_flat = labels.reshape(BATCH * NUM_ATTR)
    R = NUM_ATTR * NUM_CLASSES
    mpacked3, means_bf = pl.pallas_call(
        _prep_body,
        grid=(NUM_ATTR,),
        in_specs=[pl.BlockSpec((1, NUM_CLASSES, DIMS_PER_ATTR),
                               lambda b: (b, 0, 0))],
        out_specs=[
            pl.BlockSpec((1, NUM_CLASSES, 256), lambda b: (b, 0, 0)),
            pl.BlockSpec((1, NUM_CLASSES, DIMS_PER_ATTR),
                         lambda b: (b, 0, 0)),
        ],
        out_shape=[
            jax.ShapeDtypeStruct((NUM_ATTR, NUM_CLASSES, 256), jnp.int32),
            jax.ShapeDtypeStruct((NUM_ATTR, NUM_CLASSES, DIMS_PER_ATTR),
                                 jnp.bfloat16),
        ],
    )(means)
    mpacked = mpacked3.reshape(R, 256)

    mesh = plsc.VectorSubcoreMesh(core_axis_name="c", subcore_axis_name="s")
    sc_out = pl.kernel(
        _sc_body,
        out_type=jax.ShapeDtypeStruct((NW, L), jnp.float32),
        mesh=mesh,
        scratch_types=[
            pltpu.VMEM((RPW * NUM_ATTR,), jnp.int32),
            pltpu.VMEM((RPW * NUM_ATTR,), jnp.int32),
            pltpu.VMEM((CH, TOTAL_DIM), jnp.float32),
            pltpu.VMEM((CH, TOTAL_DIM), jnp.float32),
            pltpu.VMEM((GROWS, 256), jnp.int32),
            pltpu.VMEM((GROWS, 256), jnp.int32),
            pltpu.VMEM((L,), jnp.float32),
            pltpu.SemaphoreType.DMA,
            pltpu.SemaphoreType.DMA,
            pltpu.SemaphoreType.DMA,
            pltpu.SemaphoreType.DMA,
        ],
    )(z, lab_flat, mpacked)

    sldj2d = sldj.reshape(32, BATCH // 32)
    grid = NB_TC // BB
    off = NB_SC // BB
    tc_out = pl.pallas_call(
        _tc_body,
        grid=(grid,),
        in_specs=[
            pl.BlockSpec((BB, TOTAL_DIM), lambda b: (b + off, 0)),
            pl.BlockSpec((32, BATCH // 32), lambda b: (0, 0)),
            pl.BlockSpec((BB, NUM_ATTR), lambda b: (b + off, 0)),
            pl.BlockSpec((NUM_ATTR, NUM_CLASSES, DIMS_PER_ATTR),
                         lambda b: (0, 0, 0)),
        ],
        out_specs=pl.BlockSpec((1, 1), lambda b: (0, 0)),
        out_shape=jax.ShapeDtypeStruct((1, 1), jnp.float32),
    )(z, sldj2d, labels, means_bf)

    total = jnp.sum(sc_out) + tc_out[0, 0]
    const = 0.5 * TOTAL_DIM * np.log(2 * np.pi)
    return total / BATCH + jnp.float32(const)


# speedup vs baseline: 2.2443x; 1.0041x over previous
import jax
import jax.numpy as jnp
import numpy as np
from jax import lax
from jax.experimental import pallas as pl
from jax.experimental.pallas import tpu as pltpu
from jax.experimental.pallas import tpu_sc as plsc

NUM_ATTR = 8
NUM_CLASSES = 1000
TOTAL_DIM = 3072
DIMS_PER_ATTR = TOTAL_DIM // NUM_ATTR
BATCH = 4096

NB_SC = 2048
NB_TC = BATCH - NB_SC
BB = 512

NC, NS, L = 2, 16, 16
NW = NC * NS
RPW = NB_SC // NW
CH = 8
NCHUNK = RPW // CH
GROWS = CH * NUM_ATTR


def _sc_body(z_hbm, lab_hbm, means_hbm, out_hbm,
             lab_v, idx_v, z0, z1, g0, g1, acc_v,
             sz0, sz1, sg0, sg1):
    wid = lax.axis_index("s") * NC + lax.axis_index("c")
    row0 = wid * RPW
    bufs = ((z0, g0, sz0, sg0), (z1, g1, sz1, sg1))

    pltpu.sync_copy(lab_hbm.at[pl.ds(row0 * NUM_ATTR, RPW * NUM_ATTR)], lab_v)
    offs = (lax.iota(jnp.int32, L) % NUM_ATTR) * NUM_CLASSES

    def mkidx(j, carry):
        idx_v[pl.ds(j * L, L)] = lab_v[pl.ds(j * L, L)] + offs
        return carry

    lax.fori_loop(0, RPW * NUM_ATTR // L, mkidx, 0)

    def start(c, b):
        zb, gb, zsem, gsem = bufs[b]
        dz = pltpu.async_copy(
            z_hbm.at[pl.ds(row0 + c * CH, CH), :], zb, zsem)
        dg = pltpu.async_copy(
            means_hbm.at[idx_v.at[pl.ds(c * GROWS, GROWS)]], gb, gsem)
        return dz, dg

    pend = [start(0, 0), None]

    NACC = 8
    accs = tuple(jnp.zeros((L,), jnp.float32) for _ in range(NACC))
    for c in range(NCHUNK):
        b = c & 1
        if c + 1 < NCHUNK:
            pend[b ^ 1] = start(c + 1, b ^ 1)
        dz, dg = pend[b]
        dz.wait()
        dg.wait()
        zb, gb = bufs[b][0], bufs[b][1]

        def row(g, a, zb=zb, gb=gb):
            a = list(a)
            r = g // NUM_ATTR
            col0 = (g % NUM_ATTR) * DIMS_PER_ATTR
            for s in range(8):
                mi = gb[g, pl.ds(s * L, L)]
                m0 = lax.bitcast_convert_type(mi << 16, jnp.float32)
                m1 = lax.bitcast_convert_type(mi & jnp.int32(-65536),
                                              jnp.float32)
                d0 = zb[r, pl.ds(col0 + s * L, L)] - m0
                d1 = zb[r, pl.ds(col0 + 128 + s * L, L)] - m1
                a[(2 * s) % NACC] = a[(2 * s) % NACC] + d0 * d0
                a[(2 * s + 1) % NACC] = a[(2 * s + 1) % NACC] + d1 * d1
            for s in range(8):
                mi = gb[g, pl.ds(128 + s * L, L)]
                m2 = lax.bitcast_convert_type(mi << 16, jnp.float32)
                d2 = zb[r, pl.ds(col0 + 256 + s * L, L)] - m2
                a[s % NACC] = a[s % NACC] + d2 * d2
            return tuple(a)

        accs = lax.fori_loop(0, GROWS, row, accs)

    acc = ((accs[0] + accs[1]) + (accs[2] + accs[3])) + \
          ((accs[4] + accs[5]) + (accs[6] + accs[7]))
    acc_v[...] = 0.5 * acc
    pltpu.sync_copy(acc_v, out_hbm.at[wid])


def _prep_body(m_ref, mp_ref, mb_ref):
    m = m_ref[0]
    bits = lax.bitcast_convert_type(m, jnp.int32)
    r = jnp.int32(0x8000)
    b0 = bits[:, 0:128] + r
    b1 = bits[:, 128:256] + r
    b2 = bits[:, 256:384] + r
    w01 = ((b0 >> 16) & jnp.int32(0xFFFF)) | (b1 & jnp.int32(-65536))
    w2 = (b2 >> 16) & jnp.int32(0xFFFF)
    mp_ref[0] = jnp.concatenate([w01, w2], axis=1)
    mb_ref[0] = m.astype(jnp.bfloat16)


def _tc_body(z_ref, sldj_ref, lab_ref, means_ref, out_ref):
    pid = pl.program_id(0)

    @pl.when(pid == 0)
    def _init():
        out_ref[...] = jnp.reshape(-jnp.sum(sldj_ref[...]), (1, 1))

    z = z_ref[...]
    labs = lab_ref[...]
    class_iota = lax.broadcasted_iota(jnp.int32, (BB, NUM_CLASSES), 1)
    tacc = jnp.zeros((BB, DIMS_PER_ATTR), jnp.float32)
    for i in range(NUM_ATTR):
        onehot = (labs[:, i][:, None] == class_iota).astype(jnp.bfloat16)
        sel = jnp.dot(onehot, means_ref[i],
                      preferred_element_type=jnp.float32)
        d = z[:, i * DIMS_PER_ATTR:(i + 1) * DIMS_PER_ATTR] - sel
        tacc = tacc + d * d
    out_ref[...] += jnp.reshape(0.5 * jnp.sum(tacc), (1, 1))


@jax.jit
def kernel(z, sldj, labels, means):
    labels = labels.astype(jnp.int32)
    lab_flat = labels.reshape(BATCH * NUM_ATTR)
    R = NUM_ATTR * NUM_CLASSES
    mpacked3, means_bf = pl.pallas_call(
        _prep_body,
        grid=(NUM_ATTR,),
        in_specs=[pl.BlockSpec((1, NUM_CLASSES, DIMS_PER_ATTR),
                               lambda b: (b, 0, 0))],
        out_specs=[
            pl.BlockSpec((1, NUM_CLASSES, 256), lambda b: (b, 0, 0)),
            pl.BlockSpec((1, NUM_CLASSES, DIMS_PER_ATTR),
                         lambda b: (b, 0, 0)),
        ],
        out_shape=[
            jax.ShapeDtypeStruct((NUM_ATTR, NUM_CLASSES, 256), jnp.int32),
            jax.ShapeDtypeStruct((NUM_ATTR, NUM_CLASSES, DIMS_PER_ATTR),
                                 jnp.bfloat16),
        ],
    )(means)
    mpacked = mpacked3.reshape(R, 256)

    mesh = plsc.VectorSubcoreMesh(core_axis_name="c", subcore_axis_name="s")
    sc_out = pl.kernel(
        _sc_body,
        out_type=jax.ShapeDtypeStruct((NW, L), jnp.float32),
        mesh=mesh,
        scratch_types=[
            pltpu.VMEM((RPW * NUM_ATTR,), jnp.int32),
            pltpu.VMEM((RPW * NUM_ATTR,), jnp.int32),
            pltpu.VMEM((CH, TOTAL_DIM), jnp.float32),
            pltpu.VMEM((CH, TOTAL_DIM), jnp.float32),
            pltpu.VMEM((GROWS, 256), jnp.int32),
            pltpu.VMEM((GROWS, 256), jnp.int32),
            pltpu.VMEM((L,), jnp.float32),
            pltpu.SemaphoreType.DMA,
            pltpu.SemaphoreType.DMA,
            pltpu.SemaphoreType.DMA,
            pltpu.SemaphoreType.DMA,
        ],
    )(z, lab_flat, mpacked)

    sldj2d = sldj.reshape(32, BATCH // 32)
    grid = NB_TC // BB
    off = NB_SC // BB
    tc_out = pl.pallas_call(
        _tc_body,
        grid=(grid,),
        in_specs=[
            pl.BlockSpec((BB, TOTAL_DIM), lambda b: (b + off, 0)),
            pl.BlockSpec((32, BATCH // 32), lambda b: (0, 0)),
            pl.BlockSpec((BB, NUM_ATTR), lambda b: (b + off, 0)),
            pl.BlockSpec((NUM_ATTR, NUM_CLASSES, DIMS_PER_ATTR),
                         lambda b: (0, 0, 0)),
        ],
        out_specs=pl.BlockSpec((1, 1), lambda b: (0, 0)),
        out_shape=jax.ShapeDtypeStruct((1, 1), jnp.float32),
    )(z, sldj2d, labels, means_bf)

    total = jnp.sum(sc_out) + tc_out[0, 0]
    const = 0.5 * TOTAL_DIM * np.log(2 * np.pi)
    return total / BATCH + jnp.float32(const)


# speedup vs baseline: 2.5024x; 1.1150x over previous
import jax
import jax.numpy as jnp
import numpy as np
from jax import lax
from jax.experimental import pallas as pl
from jax.experimental.pallas import tpu as pltpu
from jax.experimental.pallas import tpu_sc as plsc

NUM_ATTR = 8
NUM_CLASSES = 1000
TOTAL_DIM = 3072
DIMS_PER_ATTR = TOTAL_DIM // NUM_ATTR
BATCH = 4096

NB_SC = 2048
NB_TC = BATCH - NB_SC
BB = 512

NC, NS, L = 2, 16, 16
NW = NC * NS
RPW = NB_SC // NW
CH = 8
NCHUNK = RPW // CH
GROWS = CH * NUM_ATTR


def _sc_body(z_hbm, lab_hbm, means_hbm, out_hbm,
             lab_v, idx_v, z0, z1, g0, g1, acc_v,
             sz0, sz1, sg0, sg1):
    wid = lax.axis_index("s") * NC + lax.axis_index("c")
    row0 = wid * RPW
    bufs = ((z0, g0, sz0, sg0), (z1, g1, sz1, sg1))

    pltpu.sync_copy(lab_hbm.at[pl.ds(row0 * NUM_ATTR, RPW * NUM_ATTR)], lab_v)
    offs = (lax.iota(jnp.int32, L) % NUM_ATTR) * NUM_CLASSES

    def mkidx(j, carry):
        idx_v[pl.ds(j * L, L)] = lab_v[pl.ds(j * L, L)] + offs
        return carry

    lax.fori_loop(0, RPW * NUM_ATTR // L, mkidx, 0)

    def start(c, b):
        zb, gb, zsem, gsem = bufs[b]
        dz = pltpu.async_copy(
            z_hbm.at[pl.ds(row0 + c * CH, CH), :], zb, zsem)
        dg = pltpu.async_copy(
            means_hbm.at[idx_v.at[pl.ds(c * GROWS, GROWS)]], gb, gsem)
        return dz, dg

    pend = [start(0, 0), None]

    NACC = 8
    accs = tuple(jnp.zeros((L,), jnp.float32) for _ in range(NACC))
    for c in range(NCHUNK):
        b = c & 1
        if c + 1 < NCHUNK:
            pend[b ^ 1] = start(c + 1, b ^ 1)
        dz, dg = pend[b]
        dz.wait()
        dg.wait()
        zb, gb = bufs[b][0], bufs[b][1]

        def row(g, a, zb=zb, gb=gb):
            a = list(a)
            r = g // NUM_ATTR
            col0 = (g % NUM_ATTR) * DIMS_PER_ATTR
            for s in range(DIMS_PER_ATTR // L):
                d = (zb[r, pl.ds(col0 + s * L, L)]
                     - gb[g, pl.ds(s * L, L)])
                a[s % NACC] = a[s % NACC] + d * d
            return tuple(a)

        accs = lax.fori_loop(0, GROWS, row, accs)

    acc = ((accs[0] + accs[1]) + (accs[2] + accs[3])) + \
          ((accs[4] + accs[5]) + (accs[6] + accs[7]))
    acc_v[...] = 0.5 * acc
    pltpu.sync_copy(acc_v, out_hbm.at[wid])


def _tc_body(z_ref, sldj_ref, lab_ref, means_ref, out_ref, mbf_ref):
    pid = pl.program_id(0)

    @pl.when(pid == 0)
    def _init():
        out_ref[...] = jnp.reshape(-jnp.sum(sldj_ref[...]), (1, 1))
        for i in range(NUM_ATTR):
            mbf_ref[i] = means_ref[i].astype(jnp.bfloat16)

    z = z_ref[...]
    labs = lab_ref[...]
    class_iota = lax.broadcasted_iota(jnp.int32, (BB, NUM_CLASSES), 1)
    tacc = jnp.zeros((BB, DIMS_PER_ATTR), jnp.float32)
    for i in range(NUM_ATTR):
        onehot = (labs[:, i][:, None] == class_iota).astype(jnp.bfloat16)
        sel = jnp.dot(onehot, mbf_ref[i],
                      preferred_element_type=jnp.float32)
        d = z[:, i * DIMS_PER_ATTR:(i + 1) * DIMS_PER_ATTR] - sel
        tacc = tacc + d * d
    out_ref[...] += jnp.reshape(0.5 * jnp.sum(tacc), (1, 1))


@jax.jit
def kernel(z, sldj, labels, means):
    labels = labels.astype(jnp.int32)
    lab_flat = labels.reshape(BATCH * NUM_ATTR)
    means2d = means.reshape(NUM_ATTR * NUM_CLASSES, DIMS_PER_ATTR)

    mesh = plsc.VectorSubcoreMesh(core_axis_name="c", subcore_axis_name="s")
    sc_out = pl.kernel(
        _sc_body,
        out_type=jax.ShapeDtypeStruct((NW, L), jnp.float32),
        mesh=mesh,
        scratch_types=[
            pltpu.VMEM((RPW * NUM_ATTR,), jnp.int32),
            pltpu.VMEM((RPW * NUM_ATTR,), jnp.int32),
            pltpu.VMEM((CH, TOTAL_DIM), jnp.float32),
            pltpu.VMEM((CH, TOTAL_DIM), jnp.float32),
            pltpu.VMEM((GROWS, DIMS_PER_ATTR), jnp.float32),
            pltpu.VMEM((GROWS, DIMS_PER_ATTR), jnp.float32),
            pltpu.VMEM((L,), jnp.float32),
            pltpu.SemaphoreType.DMA,
            pltpu.SemaphoreType.DMA,
            pltpu.SemaphoreType.DMA,
            pltpu.SemaphoreType.DMA,
        ],
    )(z, lab_flat, means2d)

    sldj2d = sldj.reshape(32, BATCH // 32)
    grid = NB_TC // BB
    off = NB_SC // BB
    tc_out = pl.pallas_call(
        _tc_body,
        grid=(grid,),
        in_specs=[
            pl.BlockSpec((BB, TOTAL_DIM), lambda b: (b + off, 0)),
            pl.BlockSpec((32, BATCH // 32), lambda b: (0, 0)),
            pl.BlockSpec((BB, NUM_ATTR), lambda b: (b + off, 0)),
            pl.BlockSpec((NUM_ATTR, NUM_CLASSES, DIMS_PER_ATTR),
                         lambda b: (0, 0, 0)),
        ],
        out_specs=pl.BlockSpec((1, 1), lambda b: (0, 0)),
        out_shape=jax.ShapeDtypeStruct((1, 1), jnp.float32),
        scratch_shapes=[pltpu.VMEM(
            (NUM_ATTR, NUM_CLASSES, DIMS_PER_ATTR), jnp.bfloat16)],
    )(z, sldj2d, labels, means)

    total = jnp.sum(sc_out) + tc_out[0, 0]
    const = 0.5 * TOTAL_DIM * np.log(2 * np.pi)
    return total / BATCH + jnp.float32(const)


# speedup vs baseline: 2.5165x; 1.0057x over previous
import jax
import jax.numpy as jnp
import numpy as np
from jax import lax
from jax.experimental import pallas as pl
from jax.experimental.pallas import tpu as pltpu
from jax.experimental.pallas import tpu_sc as plsc

NUM_ATTR = 8
NUM_CLASSES = 1000
TOTAL_DIM = 3072
DIMS_PER_ATTR = TOTAL_DIM // NUM_ATTR
BATCH = 4096

NB_SC = 1536
NB_TC = BATCH - NB_SC
BB = 512

NC, NS, L = 2, 16, 16
NW = NC * NS
RPW = NB_SC // NW
CH = 8
NCHUNK = RPW // CH
GROWS = CH * NUM_ATTR


def _sc_body(z_hbm, sldj_hbm, lab_hbm, means_hbm, out_hbm,
             lab_v, idx_v, z0, z1, g0, g1, sldj_v, acc_v,
             sz0, sz1, sg0, sg1):
    wid = lax.axis_index("s") * NC + lax.axis_index("c")
    row0 = wid * RPW
    srow0 = wid * (BATCH // NW)
    bufs = ((z0, g0, sz0, sg0), (z1, g1, sz1, sg1))

    pltpu.sync_copy(lab_hbm.at[pl.ds(row0 * NUM_ATTR, RPW * NUM_ATTR)], lab_v)
    offs = (lax.iota(jnp.int32, L) % NUM_ATTR) * NUM_CLASSES

    def mkidx(j, carry):
        idx_v[pl.ds(j * L, L)] = lab_v[pl.ds(j * L, L)] + offs
        return carry

    lax.fori_loop(0, RPW * NUM_ATTR // L, mkidx, 0)

    def start(c, b):
        zb, gb, zsem, gsem = bufs[b]
        dz = pltpu.async_copy(
            z_hbm.at[pl.ds(row0 + c * CH, CH), :], zb, zsem)
        dg = pltpu.async_copy(
            means_hbm.at[idx_v.at[pl.ds(c * GROWS, GROWS)]], gb, gsem)
        return dz, dg

    pend = [start(0, 0), None]

    pltpu.sync_copy(sldj_hbm.at[pl.ds(srow0, BATCH // NW)], sldj_v)
    sl = jnp.zeros((L,), jnp.float32)
    for k in range(BATCH // NW // L):
        sl = sl + sldj_v[pl.ds(k * L, L)]

    NACC = 8
    accs = tuple(jnp.zeros((L,), jnp.float32) for _ in range(NACC))
    for c in range(NCHUNK):
        b = c & 1
        if c + 1 < NCHUNK:
            pend[b ^ 1] = start(c + 1, b ^ 1)
        dz, dg = pend[b]
        dz.wait()
        dg.wait()
        zb, gb = bufs[b][0], bufs[b][1]

        def row(g, a, zb=zb, gb=gb):
            a = list(a)
            r = g // NUM_ATTR
            col0 = (g % NUM_ATTR) * DIMS_PER_ATTR
            for s in range(DIMS_PER_ATTR // L):
                d = (zb[r, pl.ds(col0 + s * L, L)]
                     - gb[g, pl.ds(s * L, L)])
                a[s % NACC] = a[s % NACC] + d * d
            return tuple(a)

        accs = lax.fori_loop(0, GROWS, row, accs)

    acc = ((accs[0] + accs[1]) + (accs[2] + accs[3])) + \
          ((accs[4] + accs[5]) + (accs[6] + accs[7]))
    acc_v[...] = 0.5 * acc - sl
    pltpu.sync_copy(acc_v, out_hbm.at[wid])


def _tc_body(z_ref, lab_ref, means_ref, out_ref, mbf_ref):
    pid = pl.program_id(0)

    @pl.when(pid == 0)
    def _init():
        out_ref[...] = jnp.zeros((1, 1), jnp.float32)
        for i in range(NUM_ATTR):
            mbf_ref[i] = means_ref[i].astype(jnp.bfloat16)

    z = z_ref[...]
    labs = lab_ref[...]
    class_iota = lax.broadcasted_iota(jnp.int32, (BB, NUM_CLASSES), 1)
    tacc = jnp.zeros((BB, DIMS_PER_ATTR), jnp.float32)
    for i in range(NUM_ATTR):
        onehot = (labs[:, i][:, None] == class_iota).astype(jnp.bfloat16)
        sel = jnp.dot(onehot, mbf_ref[i],
                      preferred_element_type=jnp.float32)
        d = z[:, i * DIMS_PER_ATTR:(i + 1) * DIMS_PER_ATTR] - sel
        tacc = tacc + d * d
    out_ref[...] += jnp.reshape(0.5 * jnp.sum(tacc), (1, 1))


@jax.jit
def kernel(z, sldj, labels, means):
    labels = labels.astype(jnp.int32)
    lab_flat = labels.reshape(BATCH * NUM_ATTR)
    means2d = means.reshape(NUM_ATTR * NUM_CLASSES, DIMS_PER_ATTR)

    mesh = plsc.VectorSubcoreMesh(core_axis_name="c", subcore_axis_name="s")
    sc_out = pl.kernel(
        _sc_body,
        out_type=jax.ShapeDtypeStruct((NW, L), jnp.float32),
        mesh=mesh,
        scratch_types=[
            pltpu.VMEM((RPW * NUM_ATTR,), jnp.int32),
            pltpu.VMEM((RPW * NUM_ATTR,), jnp.int32),
            pltpu.VMEM((CH, TOTAL_DIM), jnp.float32),
            pltpu.VMEM((CH, TOTAL_DIM), jnp.float32),
            pltpu.VMEM((GROWS, DIMS_PER_ATTR), jnp.float32),
            pltpu.VMEM((GROWS, DIMS_PER_ATTR), jnp.float32),
            pltpu.VMEM((BATCH // NW,), jnp.float32),
            pltpu.VMEM((L,), jnp.float32),
            pltpu.SemaphoreType.DMA,
            pltpu.SemaphoreType.DMA,
            pltpu.SemaphoreType.DMA,
            pltpu.SemaphoreType.DMA,
        ],
    )(z, sldj, lab_flat, means2d)

    grid = NB_TC // BB
    off = NB_SC // BB
    tc_out = pl.pallas_call(
        _tc_body,
        grid=(grid,),
        in_specs=[
            pl.BlockSpec((BB, TOTAL_DIM), lambda b: (b + off, 0)),
            pl.BlockSpec((BB, NUM_ATTR), lambda b: (b + off, 0)),
            pl.BlockSpec((NUM_ATTR, NUM_CLASSES, DIMS_PER_ATTR),
                         lambda b: (0, 0, 0)),
        ],
        out_specs=pl.BlockSpec((1, 1), lambda b: (0, 0)),
        out_shape=jax.ShapeDtypeStruct((1, 1), jnp.float32),
        scratch_shapes=[pltpu.VMEM(
            (NUM_ATTR, NUM_CLASSES, DIMS_PER_ATTR), jnp.bfloat16)],
    )(z, labels, means)

    total = jnp.sum(sc_out) + tc_out[0, 0]
    const = 0.5 * TOTAL_DIM * np.log(2 * np.pi)
    return total / BATCH + jnp.float32(const)
